# baseline (speedup 1.0000x reference)
import jax
import jax.numpy as jnp
from jax import lax
from jax.experimental import pallas as pl

DIM = 128
FIELDS = 26
RBLK = 16384


def _onehot_add_kernel(z_ref, noise_ref, out_ref):
    z = z_ref[...]
    iota = lax.broadcasted_iota(jnp.int32, (RBLK, DIM), 1)
    mask = (z[:, None] == iota).astype(jnp.float32)
    out_ref[...] = noise_ref[...] + mask


def kernel(z, noise):
    batch = z.shape[0]
    rows = batch * FIELDS
    z_rows = z.T.reshape(rows)
    noise_rows = jnp.transpose(noise, (1, 0, 2)).reshape(rows, DIM)
    grid = (rows // RBLK,)
    out_rows = pl.pallas_call(
        _onehot_add_kernel,
        grid=grid,
        in_specs=[
            pl.BlockSpec((RBLK,), lambda i: (i,)),
            pl.BlockSpec((RBLK, DIM), lambda i: (i, 0)),
        ],
        out_specs=pl.BlockSpec((RBLK, DIM), lambda i: (i, 0)),
        out_shape=jax.ShapeDtypeStruct((rows, DIM), jnp.float32),
    )(z_rows, noise_rows)
    out = jnp.transpose(out_rows.reshape(FIELDS, batch, DIM), (1, 0, 2))
    return (out, 0)


# speedup vs baseline: 1.0020x; 1.0020x over previous
import jax
import jax.numpy as jnp
from jax import lax
from jax.experimental import pallas as pl

DIM = 128
FIELDS = 26
RBLK = 26624


def _onehot_add_kernel(z_ref, noise_ref, out_ref):
    z = z_ref[...]
    iota = lax.broadcasted_iota(jnp.int32, (RBLK, DIM), 1)
    mask = (z[:, None] == iota).astype(jnp.float32)
    out_ref[...] = noise_ref[...] + mask


def kernel(z, noise):
    batch = z.shape[0]
    rows = batch * FIELDS
    z_rows = z.T.reshape(rows)
    noise_rows = jnp.transpose(noise, (1, 0, 2)).reshape(rows, DIM)
    grid = (rows // RBLK,)
    out_rows = pl.pallas_call(
        _onehot_add_kernel,
        grid=grid,
        in_specs=[
            pl.BlockSpec((RBLK,), lambda i: (i,)),
            pl.BlockSpec((RBLK, DIM), lambda i: (i, 0)),
        ],
        out_specs=pl.BlockSpec((RBLK, DIM), lambda i: (i, 0)),
        out_shape=jax.ShapeDtypeStruct((rows, DIM), jnp.float32),
    )(z_rows, noise_rows)
    out = jnp.transpose(out_rows.reshape(FIELDS, batch, DIM), (1, 0, 2))
    return (out, 0)


# speedup vs baseline: 1.0166x; 1.0146x over previous
import jax
import jax.numpy as jnp
from jax import lax
from jax.experimental import pallas as pl

DIM = 128
FIELDS = 26
BATCH_BLK = 1024


def _onehot_add_kernel(z_ref, noise_ref, out_ref):
    z = z_ref[...]
    iota = lax.broadcasted_iota(jnp.int32, (FIELDS, BATCH_BLK, DIM), 2)
    mask = (z[:, :, None] == iota).astype(jnp.float32)
    out_ref[...] = noise_ref[...] + mask


def kernel(z, noise):
    batch = z.shape[0]
    z_t = z.T
    noise_t = jnp.transpose(noise, (1, 0, 2))
    grid = (batch // BATCH_BLK,)
    out_t = pl.pallas_call(
        _onehot_add_kernel,
        grid=grid,
        in_specs=[
            pl.BlockSpec((FIELDS, BATCH_BLK), lambda i: (0, i)),
            pl.BlockSpec((FIELDS, BATCH_BLK, DIM), lambda i: (0, i, 0)),
        ],
        out_specs=pl.BlockSpec((FIELDS, BATCH_BLK, DIM), lambda i: (0, i, 0)),
        out_shape=jax.ShapeDtypeStruct((FIELDS, batch, DIM), jnp.float32),
    )(z_t, noise_t)
    out = jnp.transpose(out_t, (1, 0, 2))
    return (out, 0)
